# baseline (speedup 1.0000x reference)
import jax
import jax.numpy as jnp
from jax.experimental import pallas as pl
from jax.experimental.pallas import tpu as pltpu

N = 10000
IN = 256
OUT = 128
H1 = 256
TN = 3336


def _fused_encoder_kernel(
    x_ref, w1z_ref, w1h_ref, w2z_ref, w2h_ref,
    b1z_ref, b1h_ref, b2z_ref, b2h_ref, out_ref,
):
    wz1 = ((w1z_ref[0, 0] + w1z_ref[1, 0]) * 0.5).astype(jnp.bfloat16)
    wh1 = (w1h_ref[0, 0] + w1h_ref[1, 0]).astype(jnp.bfloat16)
    wz2 = ((w2z_ref[0, 0] + w2z_ref[1, 0]) * 0.25).astype(jnp.bfloat16)
    wh2 = ((w2h_ref[0, 0] + w2h_ref[1, 0]) * 0.5).astype(jnp.bfloat16)

    wc1 = jnp.concatenate([wz1, wh1], axis=1)
    wc2 = jnp.concatenate([wz2, wh2], axis=1)

    x = x_ref[...].astype(jnp.bfloat16)
    p = jnp.dot(x, wc1, preferred_element_type=jnp.float32)
    u1 = 1.0 - jnp.tanh(p[:, :H1] + 0.5 * b1z_ref[...])
    t1 = jnp.tanh(p[:, H1:] + b1h_ref[...])
    h2x = jax.nn.relu(u1 * t1).astype(jnp.bfloat16)
    q = jnp.dot(h2x, wc2, preferred_element_type=jnp.float32)
    u2 = 1.0 - jnp.tanh(q[:, :OUT] + 0.5 * b2z_ref[...])
    t2 = jnp.tanh(q[:, OUT:] + b2h_ref[...])
    out_ref[...] = 0.5 * (u2 * t2)


def kernel(x, edge_index, W1z, b1z, W1r, b1r, W1h, b1h, W2z, b2z, W2r, b2r, W2h, b2h):
    wspec1 = pl.BlockSpec((2, 1, IN, H1), lambda i: (0, 0, 0, 0))
    wspec2 = pl.BlockSpec((2, 1, H1, OUT), lambda i: (0, 0, 0, 0))
    bspec1 = pl.BlockSpec((1, H1), lambda i: (0, 0))
    bspec2 = pl.BlockSpec((1, OUT), lambda i: (0, 0))
    return pl.pallas_call(
        _fused_encoder_kernel,
        grid=(pl.cdiv(N, TN),),
        in_specs=[
            pl.BlockSpec((TN, IN), lambda i: (i, 0)),
            wspec1, wspec1, wspec2, wspec2,
            bspec1, bspec1, bspec2, bspec2,
        ],
        out_specs=pl.BlockSpec((TN, OUT), lambda i: (i, 0)),
        out_shape=jax.ShapeDtypeStruct((N, OUT), jnp.float32),
        compiler_params=pltpu.CompilerParams(
            dimension_semantics=("parallel",),
        ),
    )(
        x, W1z, W1h, W2z, W2h,
        b1z[None, :], b1h[None, :], b2z[None, :], b2h[None, :],
    )


# speedup vs baseline: 1.0175x; 1.0175x over previous
import jax
import jax.numpy as jnp
from jax.experimental import pallas as pl
from jax.experimental.pallas import tpu as pltpu

N = 10000
IN = 256
OUT = 128
H1 = 256
TN = 5000


def _fused_encoder_kernel(
    x_ref, w1z_ref, w1h_ref, w2z_ref, w2h_ref,
    b1z_ref, b1h_ref, b2z_ref, b2h_ref, out_ref,
):
    wz1 = ((w1z_ref[0, 0] + w1z_ref[1, 0]) * 0.5).astype(jnp.bfloat16)
    wh1 = (w1h_ref[0, 0] + w1h_ref[1, 0]).astype(jnp.bfloat16)
    wz2 = ((w2z_ref[0, 0] + w2z_ref[1, 0]) * 0.25).astype(jnp.bfloat16)
    wh2 = ((w2h_ref[0, 0] + w2h_ref[1, 0]) * 0.5).astype(jnp.bfloat16)

    wc1 = jnp.concatenate([wz1, wh1], axis=1)
    wc2 = jnp.concatenate([wz2, wh2], axis=1)

    x = x_ref[...].astype(jnp.bfloat16)
    p = jnp.dot(x, wc1, preferred_element_type=jnp.float32)
    u1 = 1.0 - jnp.tanh(p[:, :H1] + 0.5 * b1z_ref[...])
    t1 = jnp.tanh(p[:, H1:] + b1h_ref[...])
    h2x = jax.nn.relu(u1 * t1).astype(jnp.bfloat16)
    q = jnp.dot(h2x, wc2, preferred_element_type=jnp.float32)
    u2 = 1.0 - jnp.tanh(q[:, :OUT] + 0.5 * b2z_ref[...])
    t2 = jnp.tanh(q[:, OUT:] + b2h_ref[...])
    out_ref[...] = 0.5 * (u2 * t2)


def kernel(x, edge_index, W1z, b1z, W1r, b1r, W1h, b1h, W2z, b2z, W2r, b2r, W2h, b2h):
    wspec1 = pl.BlockSpec((2, 1, IN, H1), lambda i: (0, 0, 0, 0))
    wspec2 = pl.BlockSpec((2, 1, H1, OUT), lambda i: (0, 0, 0, 0))
    bspec1 = pl.BlockSpec((1, H1), lambda i: (0, 0))
    bspec2 = pl.BlockSpec((1, OUT), lambda i: (0, 0))
    return pl.pallas_call(
        _fused_encoder_kernel,
        grid=(pl.cdiv(N, TN),),
        in_specs=[
            pl.BlockSpec((TN, IN), lambda i: (i, 0)),
            wspec1, wspec1, wspec2, wspec2,
            bspec1, bspec1, bspec2, bspec2,
        ],
        out_specs=pl.BlockSpec((TN, OUT), lambda i: (i, 0)),
        out_shape=jax.ShapeDtypeStruct((N, OUT), jnp.float32),
        compiler_params=pltpu.CompilerParams(
            dimension_semantics=("parallel",),
        ),
    )(
        x, W1z, W1h, W2z, W2h,
        b1z[None, :], b1h[None, :], b2z[None, :], b2h[None, :],
    )
